# baseline (speedup 1.0000x reference)
import jax
import jax.numpy as jnp
from jax import lax
from jax.experimental import pallas as pl
from jax.experimental.pallas import tpu as pltpu
from jax.experimental.pallas import tpu_sc as plsc

_N_COARSE = 40962
_N_NEW = 122880
_D = 128
_LANES = 16
_NC, _NS = 2, 16
_NW = _NC * _NS

_ROWS_W = _N_NEW // _NW
_B = 128
_NCH = _ROWS_W // _B
_NPAIR = _NCH // 2
_CPY_W = _N_COARSE // _NW
_CB = 128
_NCPY = _CPY_W // _CB
_CPY_REM = _N_COARSE - _CPY_W * _NW


def _body(x, idx3, out, gbuf0, gbuf1, obuf0, obuf1, idxall,
          semg0, semg1, sems0, sems1, semc, semr):
    gbufs = (gbuf0, gbuf1)
    obufs = (obuf0, obuf1)
    semg = (semg0, semg1)
    sems = (sems0, sems1)
    cid = lax.axis_index("c")
    sid = lax.axis_index("s")
    wid = sid * _NC + cid

    pltpu.sync_copy(idx3.at[wid], idxall)

    def start_gather(c, i):
        pltpu.async_copy(x.at[idxall.at[c]], gbufs[i], semg[i])

    def wait_gather(i):
        pltpu.make_async_copy(x.at[pl.ds(0, 2 * _B)], gbufs[i], semg[i]).wait()

    def start_store(c, i):
        base = wid * _ROWS_W + c * _B
        pltpu.async_copy(obufs[i], out.at[pl.ds(_N_COARSE + base, _B)], sems[i])

    def wait_store(i):
        pltpu.make_async_copy(obufs[i], out.at[pl.ds(_N_COARSE, _B)],
                              sems[i]).wait()

    def avg(i):
        g = gbufs[i]
        o = obufs[i]

        def rows(r, cc):
            for rr in range(2):
                row = 2 * r + rr
                for v in range(_D // _LANES):
                    sl = pl.ds(v * _LANES, _LANES)
                    o[row, sl] = (g[2 * row, sl] + g[2 * row + 1, sl]) * 0.5
            return cc

        lax.fori_loop(0, _B // 2, rows, 0)

    start_gather(0, 0)
    start_gather(1, 1)

    base_c = wid * _CPY_W
    pltpu.async_copy(x.at[pl.ds(base_c, _CPY_W)],
                     out.at[pl.ds(base_c, _CPY_W)], semc)

    @pl.when(wid == _NW - 1)
    def _rem():
        pltpu.async_copy(x.at[pl.ds(_NW * _CPY_W, _CPY_REM)],
                         out.at[pl.ds(_NW * _CPY_W, _CPY_REM)], semr)

    def pair(p, carry):
        for i in range(2):
            c = 2 * p + i
            wait_gather(i)

            @pl.when(c >= 2)
            def _ws():
                wait_store(i)

            avg(i)
            start_store(c, i)

            @pl.when(p < _NPAIR - 1)
            def _ng():
                start_gather(c + 2, i)

        return carry

    lax.fori_loop(0, _NPAIR, pair, 0)
    wait_store(0)
    wait_store(1)
    pltpu.make_async_copy(x.at[pl.ds(0, _CPY_W)],
                          out.at[pl.ds(0, _CPY_W)], semc).wait()

    @pl.when(wid == _NW - 1)
    def _remw():
        pltpu.make_async_copy(x.at[pl.ds(0, _CPY_REM)],
                              out.at[pl.ds(0, _CPY_REM)], semr).wait()


@jax.jit
def kernel(x, upsample_index):
    idx3 = upsample_index.reshape(_NW, _NCH, 2 * _B)
    f = pl.kernel(
        _body,
        out_type=jax.ShapeDtypeStruct((_N_COARSE + _N_NEW, _D), jnp.float32),
        mesh=plsc.VectorSubcoreMesh(
            core_axis_name="c", subcore_axis_name="s",
            num_cores=_NC, num_subcores=_NS,
        ),
        scratch_types=[
            pltpu.VMEM((2 * _B, _D), jnp.float32),
            pltpu.VMEM((2 * _B, _D), jnp.float32),
            pltpu.VMEM((_B, _D), jnp.float32),
            pltpu.VMEM((_B, _D), jnp.float32),
            pltpu.VMEM((_NCH, 2 * _B), jnp.int32),
            pltpu.SemaphoreType.DMA,
            pltpu.SemaphoreType.DMA,
            pltpu.SemaphoreType.DMA,
            pltpu.SemaphoreType.DMA,
            pltpu.SemaphoreType.DMA,
            pltpu.SemaphoreType.DMA,
        ],
        compiler_params=pltpu.CompilerParams(use_tc_tiling_on_sc=False),
    )
    return f(x, idx3)


# speedup vs baseline: 2.0601x; 2.0601x over previous
import jax
import jax.numpy as jnp
from jax import lax
from jax.experimental import pallas as pl
from jax.experimental.pallas import tpu as pltpu
from jax.experimental.pallas import tpu_sc as plsc

_N_COARSE = 40962
_N_NEW = 122880
_D = 128
_LANES = 16
_NC, _NS = 2, 16
_NW = _NC * _NS

_ROWS_W = _N_NEW // _NW
_B = 128
_NCH = _ROWS_W // _B
_CPY_W = _N_COARSE // _NW
_NCPY = _CPY_W // _B
_NCHT = _NCH + _NCPY
_NPAIR = _NCHT // 2
_CPY_REM = _N_COARSE - _CPY_W * _NW


def _body(x, iflat, out, gbuf0, gbuf1, obuf0, obuf1, idxall,
          semg0, semg1, sems0, sems1):
    gbufs = (gbuf0, gbuf1)
    obufs = (obuf0, obuf1)
    semg = (semg0, semg1)
    sems = (sems0, sems1)
    cid = lax.axis_index("c")
    sid = lax.axis_index("s")
    wid = sid * _NC + cid

    pltpu.sync_copy(iflat.at[pl.ds(2 * wid * _ROWS_W, 2 * _B * _NCH)],
                    idxall.at[pl.ds(0, 2 * _B * _NCH)])

    half = lax.shift_right_logical(lax.iota(jnp.int32, _LANES), 1)
    cbase = wid * _CPY_W
    for k in range(_NCPY):
        for g in range(2 * _B // _LANES):
            idxall[pl.ds((_NCH + k) * 2 * _B + _LANES * g, _LANES)] = (
                cbase + k * _B + 8 * g + half)

    def out_base(c):
        return jnp.where(c < _NCH,
                         _N_COARSE + wid * _ROWS_W + c * _B,
                         cbase + (c - _NCH) * _B)

    def start_gather(c, i):
        pltpu.async_copy(x.at[idxall.at[pl.ds(2 * _B * c, 2 * _B)]],
                         gbufs[i], semg[i])

    def wait_gather(i):
        pltpu.make_async_copy(x.at[pl.ds(0, 2 * _B)], gbufs[i], semg[i]).wait()

    def start_store(c, i):
        pltpu.async_copy(obufs[i], out.at[pl.ds(out_base(c), _B)], sems[i])

    def wait_store(i):
        pltpu.make_async_copy(obufs[i], out.at[pl.ds(0, _B)], sems[i]).wait()

    def avg(i):
        g = gbufs[i]
        o = obufs[i]

        def rows(r, cc):
            for rr in range(2):
                row = 2 * r + rr
                for v in range(_D // _LANES):
                    sl = pl.ds(v * _LANES, _LANES)
                    o[row, sl] = (g[2 * row, sl] + g[2 * row + 1, sl]) * 0.5
            return cc

        lax.fori_loop(0, _B // 2, rows, 0)

    start_gather(0, 0)
    start_gather(1, 1)

    @pl.when(wid == _NW - 1)
    def _rem():
        pltpu.sync_copy(x.at[pl.ds(_NW * _CPY_W, _CPY_REM)],
                        obuf0.at[pl.ds(0, _CPY_REM)])
        pltpu.sync_copy(obuf0.at[pl.ds(0, _CPY_REM)],
                        out.at[pl.ds(_NW * _CPY_W, _CPY_REM)])

    def pair(p, carry):
        for i in range(2):
            c = 2 * p + i
            wait_gather(i)

            @pl.when(c >= 2)
            def _ws():
                wait_store(i)

            avg(i)
            start_store(c, i)

            @pl.when(p < _NPAIR - 1)
            def _ng():
                start_gather(c + 2, i)

        return carry

    lax.fori_loop(0, _NPAIR, pair, 0)
    wait_store(0)
    wait_store(1)


@jax.jit
def kernel(x, upsample_index):
    iflat = upsample_index.reshape(-1)
    f = pl.kernel(
        _body,
        out_type=jax.ShapeDtypeStruct((_N_COARSE + _N_NEW, _D), jnp.float32),
        mesh=plsc.VectorSubcoreMesh(
            core_axis_name="c", subcore_axis_name="s",
            num_cores=_NC, num_subcores=_NS,
        ),
        scratch_types=[
            pltpu.VMEM((2 * _B, _D), jnp.float32),
            pltpu.VMEM((2 * _B, _D), jnp.float32),
            pltpu.VMEM((_B, _D), jnp.float32),
            pltpu.VMEM((_B, _D), jnp.float32),
            pltpu.VMEM((2 * _B * _NCHT,), jnp.int32),
            pltpu.SemaphoreType.DMA,
            pltpu.SemaphoreType.DMA,
            pltpu.SemaphoreType.DMA,
            pltpu.SemaphoreType.DMA,
        ],
        compiler_params=pltpu.CompilerParams(use_tc_tiling_on_sc=False),
    )
    return f(x, iflat)


# speedup vs baseline: 6.9372x; 3.3674x over previous
import jax
import jax.numpy as jnp
from jax import lax
from jax.experimental import pallas as pl
from jax.experimental.pallas import tpu as pltpu
from jax.experimental.pallas import tpu_sc as plsc

_N_COARSE = 40962
_N_NEW = 122880
_D = 128
_LANES = 16
_NC, _NS = 2, 16
_NW = _NC * _NS

_ROWS_W = _N_NEW // _NW
_B = 128
_NCH = _ROWS_W // _B
_NPAIR = _NCH // 2
_CPY_W = _N_COARSE // _NW
_CB = 128
_NCPY = _CPY_W // _CB
_CPY_REM = _N_COARSE - _CPY_W * _NW


def _body(x, i0, i1, out, ga0, ga1, gb0, gb1, ob0, ob1, i0v, i1v,
          semg0, semg1, sems0, sems1, semcl0, semcl1, semcs0, semcs1):
    gas = (ga0, ga1)
    gbs = (gb0, gb1)
    obs = (ob0, ob1)
    semg = (semg0, semg1)
    sems = (sems0, sems1)
    semcl = (semcl0, semcl1)
    semcs = (semcs0, semcs1)
    cid = lax.axis_index("c")
    sid = lax.axis_index("s")
    wid = sid * _NC + cid

    pltpu.sync_copy(i0.at[pl.ds(wid * _ROWS_W, _ROWS_W)], i0v)
    pltpu.sync_copy(i1.at[pl.ds(wid * _ROWS_W, _ROWS_W)], i1v)

    def start_gather(c, i):
        sl = pl.ds(c * _B, _B)
        pltpu.async_copy(x.at[i0v.at[sl]], gas[i], semg[i])
        pltpu.async_copy(x.at[i1v.at[sl]], gbs[i], semg[i])

    def wait_gather(i):
        pltpu.make_async_copy(x.at[pl.ds(0, _B)], gas[i], semg[i]).wait()
        pltpu.make_async_copy(x.at[pl.ds(0, _B)], gbs[i], semg[i]).wait()

    def start_store(c, i):
        base = _N_COARSE + wid * _ROWS_W + c * _B
        pltpu.async_copy(obs[i], out.at[pl.ds(base, _B)], sems[i])

    def wait_store(i):
        pltpu.make_async_copy(obs[i], out.at[pl.ds(0, _B)], sems[i]).wait()

    def avg(i):
        a = gas[i]
        b = gbs[i]
        o = obs[i]

        def rows(r, cc):
            for rr in range(2):
                row = 2 * r + rr
                for v in range(_D // _LANES):
                    sl = pl.ds(v * _LANES, _LANES)
                    o[row, sl] = (a[row, sl] + b[row, sl]) * 0.5
            return cc

        lax.fori_loop(0, _B // 2, rows, 0)

    start_gather(0, 0)
    start_gather(1, 1)

    def cload(t, j):
        pltpu.async_copy(x.at[pl.ds(wid * _CPY_W + t * _CB, _CB)],
                         obs[j], semcl[j])

    def cload_wait(j):
        pltpu.make_async_copy(x.at[pl.ds(0, _CB)], obs[j], semcl[j]).wait()

    def cstore(t, j):
        pltpu.async_copy(obs[j], out.at[pl.ds(wid * _CPY_W + t * _CB, _CB)],
                         semcs[j])

    def cstore_wait(j):
        pltpu.make_async_copy(obs[j], out.at[pl.ds(0, _CB)], semcs[j]).wait()

    cload(0, 0)
    for t in range(_NCPY):
        j = t & 1
        if t + 1 < _NCPY:
            if t >= 1:
                cstore_wait(1 - j)
            cload(t + 1, 1 - j)
        cload_wait(j)
        cstore(t, j)
    cstore_wait((_NCPY - 1) & 1)
    cstore_wait((_NCPY - 2) & 1)

    @pl.when(wid == _NW - 1)
    def _rem():
        pltpu.sync_copy(x.at[pl.ds(_NW * _CPY_W, _CPY_REM)],
                        ob0.at[pl.ds(0, _CPY_REM)])
        pltpu.sync_copy(ob0.at[pl.ds(0, _CPY_REM)],
                        out.at[pl.ds(_NW * _CPY_W, _CPY_REM)])

    def pair(p, carry):
        for i in range(2):
            c = 2 * p + i
            wait_gather(i)

            @pl.when(c >= 2)
            def _ws():
                wait_store(i)

            avg(i)
            start_store(c, i)

            @pl.when(p < _NPAIR - 1)
            def _ng():
                start_gather(c + 2, i)

        return carry

    lax.fori_loop(0, _NPAIR, pair, 0)
    wait_store(0)
    wait_store(1)


@jax.jit
def kernel(x, upsample_index):
    i0 = upsample_index[:, 0]
    i1 = upsample_index[:, 1]
    f = pl.kernel(
        _body,
        out_type=jax.ShapeDtypeStruct((_N_COARSE + _N_NEW, _D), jnp.float32),
        mesh=plsc.VectorSubcoreMesh(
            core_axis_name="c", subcore_axis_name="s",
            num_cores=_NC, num_subcores=_NS,
        ),
        scratch_types=[
            pltpu.VMEM((_B, _D), jnp.float32),
            pltpu.VMEM((_B, _D), jnp.float32),
            pltpu.VMEM((_B, _D), jnp.float32),
            pltpu.VMEM((_B, _D), jnp.float32),
            pltpu.VMEM((_B, _D), jnp.float32),
            pltpu.VMEM((_B, _D), jnp.float32),
            pltpu.VMEM((_ROWS_W,), jnp.int32),
            pltpu.VMEM((_ROWS_W,), jnp.int32),
            pltpu.SemaphoreType.DMA,
            pltpu.SemaphoreType.DMA,
            pltpu.SemaphoreType.DMA,
            pltpu.SemaphoreType.DMA,
            pltpu.SemaphoreType.DMA,
            pltpu.SemaphoreType.DMA,
            pltpu.SemaphoreType.DMA,
            pltpu.SemaphoreType.DMA,
        ],
        compiler_params=pltpu.CompilerParams(use_tc_tiling_on_sc=False),
    )
    return f(x, i0, i1)
